# Initial kernel scaffold; baseline (speedup 1.0000x reference)
import jax
import jax.numpy as jnp
from jax.experimental import pallas as pl


def kernel(x, edge_index, edge_attr, params):
    raise NotImplementedError("write your pallas kernel here")


# speedup vs baseline: 1.8660x; 1.8660x over previous
import functools

import jax
import jax.numpy as jnp
from jax import lax
from jax.experimental import pallas as pl
from jax.experimental.pallas import tpu as pltpu
from jax.experimental.pallas import tpu_sc as plsc

NC = 2
NS = 16
NW = NC * NS
L = 16
CH = 80


def _dot(a, b):
    return lax.dot_general(a, b, (((1,), (0,)), ((), ())),
                           precision=lax.Precision.HIGHEST,
                           preferred_element_type=jnp.float32)


def _ln(y, g, b):
    mu = jnp.mean(y, axis=-1, keepdims=True)
    var = jnp.mean((y - mu) ** 2, axis=-1, keepdims=True)
    return (y - mu) / jnp.sqrt(var + 1e-5) * g + b


def _pre_body(x_ref, a_ref, b_ref, p_ref, q_ref):
    xb = x_ref[...]
    p_ref[...] = _dot(xb, a_ref[...])
    q_ref[...] = _dot(xb, b_ref[...])


def _tc_pre(x, A, B, bn=1000):
    n, d = x.shape
    return pl.pallas_call(
        _pre_body,
        grid=(n // bn,),
        in_specs=[pl.BlockSpec((bn, d), lambda i: (i, 0)),
                  pl.BlockSpec((d, d), lambda i: (0, 0)),
                  pl.BlockSpec((d, d), lambda i: (0, 0))],
        out_specs=[pl.BlockSpec((bn, d), lambda i: (i, 0)),
                   pl.BlockSpec((bn, d), lambda i: (i, 0))],
        out_shape=[jax.ShapeDtypeStruct((n, d), jnp.float32)] * 2,
        compiler_params=pltpu.CompilerParams(
            dimension_semantics=("parallel",)),
    )(x, A, B)


def _edge_mlp_body(s_ref, e_ref, c_ref, b0_ref, w1_ref, b1_ref, w2_ref,
                   b2_ref, g_ref, bb_ref, o_ref):
    h = s_ref[...] + _dot(e_ref[...], c_ref[...]) + b0_ref[...]
    h = jnp.maximum(h, 0.0)
    h = jnp.maximum(_dot(h, w1_ref[...]) + b1_ref[...], 0.0)
    y = _dot(h, w2_ref[...]) + b2_ref[...]
    o_ref[...] = _ln(y, g_ref[...], bb_ref[...])


def _tc_edge_mlp(S, e, C, b0, W1, b1, W2, b2, g, bb, be=512):
    E, d = S.shape
    wspec = pl.BlockSpec((d, d), lambda i: (0, 0))
    vspec = pl.BlockSpec((1, d), lambda i: (0, 0))
    return pl.pallas_call(
        _edge_mlp_body,
        grid=(E // be,),
        in_specs=[pl.BlockSpec((be, d), lambda i: (i, 0)),
                  pl.BlockSpec((be, d), lambda i: (i, 0)),
                  wspec, vspec, wspec, vspec, wspec, vspec, vspec, vspec],
        out_specs=pl.BlockSpec((be, d), lambda i: (i, 0)),
        out_shape=jax.ShapeDtypeStruct((E, d), jnp.float32),
        compiler_params=pltpu.CompilerParams(
            dimension_semantics=("parallel",)),
    )(S, e, C, b0, W1, b1, W2, b2, g, bb)


def _node_body(x_ref, p0_ref, p1_ref, ua_ref, ub_ref, b0_ref, w1_ref, b1_ref,
               w2_ref, b2_ref, g_ref, bb_ref, o_ref):
    aggr = p0_ref[...] / jnp.maximum(p1_ref[...], 1.0)
    h = _dot(x_ref[...], ua_ref[...]) + _dot(aggr, ub_ref[...]) + b0_ref[...]
    h = jnp.maximum(h, 0.0)
    h = jnp.maximum(_dot(h, w1_ref[...]) + b1_ref[...], 0.0)
    y = _dot(h, w2_ref[...]) + b2_ref[...]
    o_ref[...] = _ln(y, g_ref[...], bb_ref[...])


def _tc_node(x, p0, p1, Ua, Ub, b0, W1, b1, W2, b2, g, bb, bn=1000):
    n, d = x.shape
    wspec = pl.BlockSpec((d, d), lambda i: (0, 0))
    vspec = pl.BlockSpec((1, d), lambda i: (0, 0))
    return pl.pallas_call(
        _node_body,
        grid=(n // bn,),
        in_specs=[pl.BlockSpec((bn, d), lambda i: (i, 0)),
                  pl.BlockSpec((bn, d), lambda i: (i, 0)),
                  pl.BlockSpec((bn, d), lambda i: (i, 0)),
                  wspec, wspec, vspec, wspec, vspec, wspec, vspec, vspec,
                  vspec],
        out_specs=pl.BlockSpec((bn, d), lambda i: (i, 0)),
        out_shape=jax.ShapeDtypeStruct((n, d), jnp.float32),
        compiler_params=pltpu.CompilerParams(
            dimension_semantics=("parallel",)),
    )(x, p0, p1, Ua, Ub, b0, W1, b1, W2, b2, g, bb)


def _sc_gather_sum(P, Q, src3, dst3):
    n, d = P.shape
    nch = src3.shape[1]
    E = NW * nch * CH
    mesh = plsc.VectorSubcoreMesh(core_axis_name="c", subcore_axis_name="s",
                                  num_cores=NC, num_subcores=NS)

    @functools.partial(
        pl.kernel,
        out_type=jax.ShapeDtypeStruct((E, d), jnp.float32),
        mesh=mesh,
        scratch_types=[
            pltpu.VMEM((nch, CH), jnp.int32),
            pltpu.VMEM((nch, CH), jnp.int32),
            pltpu.VMEM((CH, d), jnp.float32),
            pltpu.VMEM((CH, d), jnp.float32),
            pltpu.SemaphoreType.DMA,
            pltpu.SemaphoreType.DMA,
        ],
    )
    def k(p_hbm, q_hbm, src_hbm, dst_hbm, out_hbm, di_v, si_v, bufd, bufs,
          sem1, sem2):
        wid = lax.axis_index("s") * NC + lax.axis_index("c")
        base = wid * (nch * CH)
        pltpu.sync_copy(dst_hbm.at[wid], di_v)
        pltpu.sync_copy(src_hbm.at[wid], si_v)

        def chunk(c, carry):
            cp1 = pltpu.async_copy(p_hbm.at[di_v.at[c]], bufd, sem1)
            cp2 = pltpu.async_copy(q_hbm.at[si_v.at[c]], bufs, sem2)
            cp1.wait()
            cp2.wait()

            def row(r, carry2):
                for j in range(d // L):
                    sl = pl.ds(j * L, L)
                    bufd[r, sl] = bufd[r, sl] + bufs[r, sl]
                return carry2

            lax.fori_loop(0, CH, row, 0)
            pltpu.sync_copy(bufd, out_hbm.at[pl.ds(base + c * CH, CH)])
            return carry

        lax.fori_loop(0, nch, chunk, 0)

    return k(P, Q, src3, dst3)


def _sc_scatter_add(m, dst3, n):
    E, d = m.shape
    nch = dst3.shape[1]
    zc = CH
    nzc = n // zc
    kmax = -(-nzc // NS)
    mesh = plsc.VectorSubcoreMesh(core_axis_name="c", subcore_axis_name="s",
                                  num_cores=NC, num_subcores=NS)

    @functools.partial(
        pl.kernel,
        out_type=jax.ShapeDtypeStruct((NC, n, d), jnp.float32),
        mesh=mesh,
        scratch_types=[
            pltpu.VMEM((nch, CH), jnp.int32),
            pltpu.VMEM((CH, d), jnp.float32),
            pltpu.VMEM_SHARED((n, d), jnp.float32),
        ],
    )
    def k(m_hbm, dst_hbm, out_hbm, di_v, buf, acc):
        cid = lax.axis_index("c")
        sid = lax.axis_index("s")
        base = sid * (nch * CH)

        def fill(val):
            def frow(r, carry):
                for j in range(d // L):
                    buf[r, pl.ds(j * L, L)] = jnp.full((L,), val, jnp.float32)
                return carry

            lax.fori_loop(0, CH, frow, 0)

        fill(0.0)
        for kk in range(kmax):
            c = sid + kk * NS

            @pl.when(c < nzc)
            def _():
                pltpu.sync_copy(buf, acc.at[pl.ds(c * zc, zc)])

        plsc.subcore_barrier()
        pltpu.sync_copy(dst_hbm.at[sid], di_v)

        @pl.when(cid == 0)
        def _():
            def chunk(c, carry):
                pltpu.sync_copy(m_hbm.at[pl.ds(base + c * CH, CH)], buf)
                pltpu.sync_copy(buf, acc.at[di_v.at[c]], add=True)
                return carry

            lax.fori_loop(0, nch, chunk, 0)

        @pl.when(cid == 1)
        def _():
            fill(1.0)

            def chunk(c, carry):
                pltpu.sync_copy(buf, acc.at[di_v.at[c]], add=True)
                return carry

            lax.fori_loop(0, nch, chunk, 0)

        plsc.subcore_barrier()
        for kk in range(kmax):
            c = sid + kk * NS

            @pl.when(c < nzc)
            def _():
                pltpu.sync_copy(acc.at[pl.ds(c * zc, zc)],
                                out_hbm.at[cid, pl.ds(c * zc, zc)])

    return k(m, dst3)


def kernel(x, edge_index, edge_attr, params):
    n, d = x.shape
    E = edge_index.shape[1]
    assert d == 128 and E % (NW * CH) == 0 and n % (NS * 5) == 0

    src = edge_index[0].astype(jnp.int32)
    dst = edge_index[1].astype(jnp.int32)
    ew = E // NW
    src3 = src.reshape(NW, ew // CH, CH)
    dst3 = dst.reshape(NW, ew // CH, CH)
    dst3s = dst.reshape(NS, (E // NS) // CH, CH)

    pm, pn = params["msg"], params["node"]
    W0, W1, W2 = pm["Ws"]
    b0, b1, b2 = [b.reshape(1, d) for b in pm["bs"]]
    g, bb = pm["g"].reshape(1, d), pm["b"].reshape(1, d)
    A, B, C = W0[:d], W0[d:2 * d], W0[2 * d:]

    P, Q = _tc_pre(x, A, B)
    S = _sc_gather_sum(P, Q, src3, dst3)
    m_ext = _tc_edge_mlp(S, edge_attr, C, b0, W1, b1, W2, b2, g, bb)
    partials = _sc_scatter_add(m_ext, dst3s, n)

    U0, V1, V2 = pn["Ws"]
    c0, c1, c2 = [b.reshape(1, d) for b in pn["bs"]]
    gn, bn = pn["g"].reshape(1, d), pn["b"].reshape(1, d)
    Ua, Ub = U0[:d], U0[d:]
    x_out = _tc_node(x, partials[0], partials[1], Ua, Ub, c0, V1, c1, V2, c2,
                     gn, bn)
    return (x_out, edge_attr)


# speedup vs baseline: 2.1439x; 1.1489x over previous
import functools

import jax
import jax.numpy as jnp
from jax import lax
from jax.experimental import pallas as pl
from jax.experimental.pallas import tpu as pltpu
from jax.experimental.pallas import tpu_sc as plsc

NC = 2
NS = 16
NW = NC * NS
L = 16
CH = 40
G = 5


def _dot(a, b):
    return lax.dot_general(a, b, (((1,), (0,)), ((), ())),
                           precision=lax.Precision.HIGHEST,
                           preferred_element_type=jnp.float32)


def _ln(y, g, b):
    mu = jnp.mean(y, axis=-1, keepdims=True)
    var = jnp.mean((y - mu) ** 2, axis=-1, keepdims=True)
    return (y - mu) / jnp.sqrt(var + 1e-5) * g + b


def _pre_body(x_ref, a_ref, b_ref, p_ref, q_ref):
    xb = x_ref[...]
    p_ref[...] = _dot(xb, a_ref[...])
    q_ref[...] = _dot(xb, b_ref[...])


def _tc_pre(x, A, B, bn=1000):
    n, d = x.shape
    return pl.pallas_call(
        _pre_body,
        grid=(n // bn,),
        in_specs=[pl.BlockSpec((bn, d), lambda i: (i, 0)),
                  pl.BlockSpec((d, d), lambda i: (0, 0)),
                  pl.BlockSpec((d, d), lambda i: (0, 0))],
        out_specs=[pl.BlockSpec((bn, d), lambda i: (i, 0)),
                   pl.BlockSpec((bn, d), lambda i: (i, 0))],
        out_shape=[jax.ShapeDtypeStruct((n, d), jnp.float32)] * 2,
        compiler_params=pltpu.CompilerParams(
            dimension_semantics=("parallel",)),
    )(x, A, B)


def _edge_mlp_body(s_ref, e_ref, c_ref, b0_ref, w1_ref, b1_ref, w2_ref,
                   b2_ref, g_ref, bb_ref, o_ref):
    h = s_ref[...] + _dot(e_ref[...], c_ref[...]) + b0_ref[...]
    h = jnp.maximum(h, 0.0)
    h = jnp.maximum(_dot(h, w1_ref[...]) + b1_ref[...], 0.0)
    y = _dot(h, w2_ref[...]) + b2_ref[...]
    o_ref[...] = _ln(y, g_ref[...], bb_ref[...])


def _tc_edge_mlp(S, e, C, b0, W1, b1, W2, b2, g, bb, be=512):
    E, d = S.shape
    wspec = pl.BlockSpec((d, d), lambda i: (0, 0))
    vspec = pl.BlockSpec((1, d), lambda i: (0, 0))
    return pl.pallas_call(
        _edge_mlp_body,
        grid=(E // be,),
        in_specs=[pl.BlockSpec((be, d), lambda i: (i, 0)),
                  pl.BlockSpec((be, d), lambda i: (i, 0)),
                  wspec, vspec, wspec, vspec, wspec, vspec, vspec, vspec],
        out_specs=pl.BlockSpec((be, d), lambda i: (i, 0)),
        out_shape=jax.ShapeDtypeStruct((E, d), jnp.float32),
        compiler_params=pltpu.CompilerParams(
            dimension_semantics=("parallel",)),
    )(S, e, C, b0, W1, b1, W2, b2, g, bb)


def _node_body(x_ref, p0_ref, p1_ref, ua_ref, ub_ref, b0_ref, w1_ref, b1_ref,
               w2_ref, b2_ref, g_ref, bb_ref, o_ref):
    aggr = p0_ref[...] / jnp.maximum(p1_ref[...], 1.0)
    h = _dot(x_ref[...], ua_ref[...]) + _dot(aggr, ub_ref[...]) + b0_ref[...]
    h = jnp.maximum(h, 0.0)
    h = jnp.maximum(_dot(h, w1_ref[...]) + b1_ref[...], 0.0)
    y = _dot(h, w2_ref[...]) + b2_ref[...]
    o_ref[...] = _ln(y, g_ref[...], bb_ref[...])


def _tc_node(x, p0, p1, Ua, Ub, b0, W1, b1, W2, b2, g, bb, bn=1000):
    n, d = x.shape
    wspec = pl.BlockSpec((d, d), lambda i: (0, 0))
    vspec = pl.BlockSpec((1, d), lambda i: (0, 0))
    return pl.pallas_call(
        _node_body,
        grid=(n // bn,),
        in_specs=[pl.BlockSpec((bn, d), lambda i: (i, 0)),
                  pl.BlockSpec((bn, d), lambda i: (i, 0)),
                  pl.BlockSpec((bn, d), lambda i: (i, 0)),
                  wspec, wspec, vspec, wspec, vspec, wspec, vspec, vspec,
                  vspec],
        out_specs=pl.BlockSpec((bn, d), lambda i: (i, 0)),
        out_shape=jax.ShapeDtypeStruct((n, d), jnp.float32),
        compiler_params=pltpu.CompilerParams(
            dimension_semantics=("parallel",)),
    )(x, p0, p1, Ua, Ub, b0, W1, b1, W2, b2, g, bb)


def _sc_gather_sum(P, Q, src2, dst2):
    n, d = P.shape
    ew = src2.shape[1]
    nch = ew // CH
    E = NW * ew
    mesh = plsc.VectorSubcoreMesh(core_axis_name="c", subcore_axis_name="s",
                                  num_cores=NC, num_subcores=NS)

    @functools.partial(
        pl.kernel,
        out_type=jax.ShapeDtypeStruct((E, d), jnp.float32),
        mesh=mesh,
        scratch_types=(
            [pltpu.VMEM((ew,), jnp.int32)] * 2
            + [pltpu.VMEM((CH, d), jnp.float32)] * (3 * G)
            + [pltpu.SemaphoreType.DMA] * (2 * G)
        ),
    )
    def k(p_hbm, q_hbm, src_hbm, dst_hbm, out_hbm, *sc):
        di_v, si_v = sc[0], sc[1]
        bufd = sc[2:2 + G]
        bufq = sc[2 + G:2 + 2 * G]
        sbuf = sc[2 + 2 * G:2 + 3 * G]
        gsem = sc[2 + 3 * G:2 + 4 * G]
        wsem = sc[2 + 4 * G:2 + 5 * G]
        wid = lax.axis_index("s") * NC + lax.axis_index("c")
        base = wid * ew
        pltpu.sync_copy(dst_hbm.at[wid], di_v)
        pltpu.sync_copy(src_hbm.at[wid], si_v)

        def fire(b, c):
            pltpu.async_copy(p_hbm.at[di_v.at[pl.ds(c * CH, CH)]], bufd[b],
                             gsem[b])
            pltpu.async_copy(q_hbm.at[si_v.at[pl.ds(c * CH, CH)]], bufq[b],
                             gsem[b])

        for b in range(G):
            fire(b, b)

        def body(kk, carry):
            for b in range(G):
                c = kk * G + b
                pltpu.make_async_copy(p_hbm.at[di_v.at[pl.ds(0, CH)]],
                                      bufd[b], gsem[b]).wait()
                pltpu.make_async_copy(q_hbm.at[si_v.at[pl.ds(0, CH)]],
                                      bufq[b], gsem[b]).wait()

                @pl.when(kk > 0)
                def _():
                    pltpu.make_async_copy(
                        sbuf[b], out_hbm.at[pl.ds(base, CH)], wsem[b]).wait()

                def row(r, carry2):
                    for j in range(d // L):
                        sl = pl.ds(j * L, L)
                        sbuf[b][r, sl] = bufd[b][r, sl] + bufq[b][r, sl]
                    return carry2

                lax.fori_loop(0, CH, row, 0)
                pltpu.async_copy(sbuf[b],
                                 out_hbm.at[pl.ds(base + c * CH, CH)],
                                 wsem[b])

                @pl.when(c + G < nch)
                def _():
                    fire(b, c + G)
            return carry

        lax.fori_loop(0, nch // G, body, 0)
        for b in range(G):
            pltpu.make_async_copy(sbuf[b], out_hbm.at[pl.ds(base, CH)],
                                  wsem[b]).wait()

    return k(P, Q, src2, dst2)


def _sc_scatter_add(m, dst3, n):
    E, d = m.shape
    nch = dst3.shape[1]
    zc = CH
    nzc = n // zc
    kmax = -(-nzc // NS)
    mesh = plsc.VectorSubcoreMesh(core_axis_name="c", subcore_axis_name="s",
                                  num_cores=NC, num_subcores=NS)

    @functools.partial(
        pl.kernel,
        out_type=jax.ShapeDtypeStruct((NC, n, d), jnp.float32),
        mesh=mesh,
        scratch_types=[
            pltpu.VMEM((CH,), jnp.int32),
            pltpu.VMEM((CH,), jnp.int32),
            pltpu.VMEM((CH, d), jnp.float32),
            pltpu.VMEM((CH, d), jnp.float32),
            pltpu.SemaphoreType.DMA,
            pltpu.SemaphoreType.DMA,
            pltpu.VMEM_SHARED((n, d), jnp.float32),
        ],
    )
    def k(m_hbm, dst_hbm, out_hbm, ib0, ib1, buf0, buf1, sem0, sem1, acc):
        cid = lax.axis_index("c")
        sid = lax.axis_index("s")
        base = sid * (nch * CH)
        ibs = (ib0, ib1)
        bufs = (buf0, buf1)
        sems = (sem0, sem1)

        def fill(ref, val):
            def frow(r, carry):
                for j in range(d // L):
                    ref[r, pl.ds(j * L, L)] = jnp.full((L,), val, jnp.float32)
                return carry

            lax.fori_loop(0, CH, frow, 0)

        fill(buf0, 0.0)
        for kk in range(kmax):
            c = sid + kk * NS

            @pl.when(c < nzc)
            def _():
                pltpu.sync_copy(buf0, acc.at[pl.ds(c * zc, zc)])

        plsc.subcore_barrier()

        def fire_idx(b, c):
            pltpu.async_copy(dst_hbm.at[sid, c], ibs[b], sems[b])

        def drain_idx(b):
            pltpu.make_async_copy(dst_hbm.at[sid, 0], ibs[b], sems[b]).wait()

        @pl.when(cid == 0)
        def _():
            for b in range(2):
                fire_idx(b, b)
                pltpu.async_copy(m_hbm.at[pl.ds(base + b * CH, CH)], bufs[b],
                                 sems[b])

            def pair(kk, carry):
                for b in range(2):
                    c = 2 * kk + b
                    drain_idx(b)
                    pltpu.make_async_copy(
                        m_hbm.at[pl.ds(base, CH)], bufs[b], sems[b]).wait()
                    pltpu.sync_copy(bufs[b], acc.at[ibs[b]], add=True)

                    @pl.when(c + 2 < nch)
                    def _():
                        fire_idx(b, c + 2)
                        pltpu.async_copy(
                            m_hbm.at[pl.ds(base + (c + 2) * CH, CH)],
                            bufs[b], sems[b])
                return carry

            lax.fori_loop(0, nch // 2, pair, 0)

        @pl.when(cid == 1)
        def _():
            fill(buf1, 1.0)
            for b in range(2):
                fire_idx(b, b)

            def pair(kk, carry):
                for b in range(2):
                    c = 2 * kk + b
                    drain_idx(b)
                    pltpu.sync_copy(buf1, acc.at[ibs[b]], add=True)

                    @pl.when(c + 2 < nch)
                    def _():
                        fire_idx(b, c + 2)
                return carry

            lax.fori_loop(0, nch // 2, pair, 0)

        plsc.subcore_barrier()
        for kk in range(kmax):
            c = sid + kk * NS

            @pl.when(c < nzc)
            def _():
                pltpu.sync_copy(acc.at[pl.ds(c * zc, zc)],
                                out_hbm.at[cid, pl.ds(c * zc, zc)])

    return k(m, dst3)


def kernel(x, edge_index, edge_attr, params):
    n, d = x.shape
    E = edge_index.shape[1]
    assert d == 128 and E % (NW * CH) == 0 and n % (NS * 5) == 0

    src = edge_index[0].astype(jnp.int32)
    dst = edge_index[1].astype(jnp.int32)
    ew = E // NW
    src2 = src.reshape(NW, ew)
    dst2 = dst.reshape(NW, ew)
    dst3s = dst.reshape(NS, (E // NS) // CH, CH)

    pm, pn = params["msg"], params["node"]
    W0, W1, W2 = pm["Ws"]
    b0, b1, b2 = [b.reshape(1, d) for b in pm["bs"]]
    g, bb = pm["g"].reshape(1, d), pm["b"].reshape(1, d)
    A, B, C = W0[:d], W0[d:2 * d], W0[2 * d:]

    P, Q = _tc_pre(x, A, B)
    S = _sc_gather_sum(P, Q, src2, dst2)
    m_ext = _tc_edge_mlp(S, edge_attr, C, b0, W1, b1, W2, b2, g, bb)
    partials = _sc_scatter_add(m_ext, dst3s, n)

    U0, V1, V2 = pn["Ws"]
    c0, c1, c2 = [b.reshape(1, d) for b in pn["bs"]]
    gn, bn = pn["g"].reshape(1, d), pn["b"].reshape(1, d)
    Ua, Ub = U0[:d], U0[d:]
    x_out = _tc_node(x, partials[0], partials[1], Ua, Ub, c0, V1, c1, V2, c2,
                     gn, bn)
    return (x_out, edge_attr)


# speedup vs baseline: 3.7493x; 1.7488x over previous
import functools

import jax
import jax.numpy as jnp
from jax import lax
from jax.experimental import pallas as pl
from jax.experimental.pallas import tpu as pltpu
from jax.experimental.pallas import tpu_sc as plsc

NC = 2
NS = 16
NW = NC * NS
L = 16
CH = 40
G = 5


def _dot(a, b):
    return lax.dot_general(a, b, (((1,), (0,)), ((), ())),
                           precision=lax.Precision.DEFAULT,
                           preferred_element_type=jnp.float32)


def _ln(y, g, b):
    mu = jnp.mean(y, axis=-1, keepdims=True)
    var = jnp.mean((y - mu) ** 2, axis=-1, keepdims=True)
    return (y - mu) / jnp.sqrt(var + 1e-5) * g + b


def _pre_body(x_ref, a_ref, b_ref, p_ref, q_ref):
    xb = x_ref[...]
    p_ref[...] = _dot(xb, a_ref[...])
    q_ref[...] = _dot(xb, b_ref[...])


def _tc_pre(x, A, B, bn=1000):
    n, d = x.shape
    return pl.pallas_call(
        _pre_body,
        grid=(n // bn,),
        in_specs=[pl.BlockSpec((bn, d), lambda i: (i, 0)),
                  pl.BlockSpec((d, d), lambda i: (0, 0)),
                  pl.BlockSpec((d, d), lambda i: (0, 0))],
        out_specs=[pl.BlockSpec((bn, d), lambda i: (i, 0)),
                   pl.BlockSpec((bn, d), lambda i: (i, 0))],
        out_shape=[jax.ShapeDtypeStruct((n, d), jnp.float32)] * 2,
        compiler_params=pltpu.CompilerParams(
            dimension_semantics=("parallel",)),
    )(x, A, B)


def _edge_mlp_body(s_ref, e_ref, c_ref, b0_ref, w1_ref, b1_ref, w2_ref,
                   b2_ref, g_ref, bb_ref, o_ref):
    h = s_ref[...] + _dot(e_ref[...], c_ref[...]) + b0_ref[...]
    h = jnp.maximum(h, 0.0)
    h = jnp.maximum(_dot(h, w1_ref[...]) + b1_ref[...], 0.0)
    y = _dot(h, w2_ref[...]) + b2_ref[...]
    o_ref[...] = _ln(y, g_ref[...], bb_ref[...])


def _tc_edge_mlp(S, e, C, b0, W1, b1, W2, b2, g, bb, be=1000):
    E, d = S.shape
    wspec = pl.BlockSpec((d, d), lambda i: (0, 0))
    vspec = pl.BlockSpec((1, d), lambda i: (0, 0))
    return pl.pallas_call(
        _edge_mlp_body,
        grid=(E // be,),
        in_specs=[pl.BlockSpec((be, d), lambda i: (i, 0)),
                  pl.BlockSpec((be, d), lambda i: (i, 0)),
                  wspec, vspec, wspec, vspec, wspec, vspec, vspec, vspec],
        out_specs=pl.BlockSpec((be, d), lambda i: (i, 0)),
        out_shape=jax.ShapeDtypeStruct((E, d), jnp.float32),
        compiler_params=pltpu.CompilerParams(
            dimension_semantics=("parallel",)),
    )(S, e, C, b0, W1, b1, W2, b2, g, bb)


def _node_body(x_ref, s0_ref, c0_ref, s1_ref, c1_ref, ua_ref, ub_ref, b0_ref,
               w1_ref, b1_ref, w2_ref, b2_ref, g_ref, bb_ref, o_ref):
    s = s0_ref[...] + s1_ref[...]
    cnt = c0_ref[...] + c1_ref[...]
    aggr = s / jnp.maximum(cnt, 1.0)
    h = _dot(x_ref[...], ua_ref[...]) + _dot(aggr, ub_ref[...]) + b0_ref[...]
    h = jnp.maximum(h, 0.0)
    h = jnp.maximum(_dot(h, w1_ref[...]) + b1_ref[...], 0.0)
    y = _dot(h, w2_ref[...]) + b2_ref[...]
    o_ref[...] = _ln(y, g_ref[...], bb_ref[...])


def _tc_node(x, parts, Ua, Ub, b0, W1, b1, W2, b2, g, bb, bn=1000):
    n, d = x.shape
    wspec = pl.BlockSpec((d, d), lambda i: (0, 0))
    vspec = pl.BlockSpec((1, d), lambda i: (0, 0))
    nspec = pl.BlockSpec((bn, d), lambda i: (i, 0))
    return pl.pallas_call(
        _node_body,
        grid=(n // bn,),
        in_specs=[nspec, nspec, nspec, nspec, nspec,
                  wspec, wspec, vspec, wspec, vspec, wspec, vspec, vspec,
                  vspec],
        out_specs=nspec,
        out_shape=jax.ShapeDtypeStruct((n, d), jnp.float32),
        compiler_params=pltpu.CompilerParams(
            dimension_semantics=("parallel",)),
    )(x, parts[0][0], parts[0][1], parts[1][0], parts[1][1],
      Ua, Ub, b0, W1, b1, W2, b2, g, bb)


def _sc_gather_sum(P, Q, src2, dst2):
    n, d = P.shape
    ew = src2.shape[1]
    nch = ew // CH
    E = NW * ew
    mesh = plsc.VectorSubcoreMesh(core_axis_name="c", subcore_axis_name="s",
                                  num_cores=NC, num_subcores=NS)

    @functools.partial(
        pl.kernel,
        out_type=jax.ShapeDtypeStruct((E, d), jnp.float32),
        mesh=mesh,
        scratch_types=(
            [pltpu.VMEM((ew,), jnp.int32)] * 2
            + [pltpu.VMEM((CH, d), jnp.float32)] * (3 * G)
            + [pltpu.SemaphoreType.DMA] * (2 * G)
        ),
    )
    def k(p_hbm, q_hbm, src_hbm, dst_hbm, out_hbm, *sc):
        di_v, si_v = sc[0], sc[1]
        bufd = sc[2:2 + G]
        bufq = sc[2 + G:2 + 2 * G]
        sbuf = sc[2 + 2 * G:2 + 3 * G]
        gsem = sc[2 + 3 * G:2 + 4 * G]
        wsem = sc[2 + 4 * G:2 + 5 * G]
        wid = lax.axis_index("s") * NC + lax.axis_index("c")
        base = wid * ew
        pltpu.sync_copy(dst_hbm.at[wid], di_v)
        pltpu.sync_copy(src_hbm.at[wid], si_v)

        def fire(b, c):
            pltpu.async_copy(p_hbm.at[di_v.at[pl.ds(c * CH, CH)]], bufd[b],
                             gsem[b])
            pltpu.async_copy(q_hbm.at[si_v.at[pl.ds(c * CH, CH)]], bufq[b],
                             gsem[b])

        for b in range(G):
            fire(b, b)

        def body(kk, carry):
            for b in range(G):
                c = kk * G + b
                pltpu.make_async_copy(p_hbm.at[di_v.at[pl.ds(0, CH)]],
                                      bufd[b], gsem[b]).wait()
                pltpu.make_async_copy(q_hbm.at[si_v.at[pl.ds(0, CH)]],
                                      bufq[b], gsem[b]).wait()

                @pl.when(kk > 0)
                def _():
                    pltpu.make_async_copy(
                        sbuf[b], out_hbm.at[pl.ds(base, CH)], wsem[b]).wait()

                def row(r, carry2):
                    for j in range(d // L):
                        sl = pl.ds(j * L, L)
                        sbuf[b][r, sl] = bufd[b][r, sl] + bufq[b][r, sl]
                    return carry2

                lax.fori_loop(0, CH, row, 0)
                pltpu.async_copy(sbuf[b],
                                 out_hbm.at[pl.ds(base + c * CH, CH)],
                                 wsem[b])

                @pl.when(c + G < nch)
                def _():
                    fire(b, c + G)
            return carry

        lax.fori_loop(0, nch // G, body, 0)
        for b in range(G):
            pltpu.make_async_copy(sbuf[b], out_hbm.at[pl.ds(base, CH)],
                                  wsem[b]).wait()

    return k(P, Q, src2, dst2)


def _sc_scatter_add(m, dst3, n):
    E, d = m.shape
    nch = dst3.shape[1]
    zc = CH
    nzc = n // zc
    kmax = -(-nzc // NS)
    mesh = plsc.VectorSubcoreMesh(core_axis_name="c", subcore_axis_name="s",
                                  num_cores=NC, num_subcores=NS)

    @functools.partial(
        pl.kernel,
        out_type=jax.ShapeDtypeStruct((NC, n, d), jnp.float32),
        mesh=mesh,
        scratch_types=[
            pltpu.VMEM((CH,), jnp.int32),
            pltpu.VMEM((CH,), jnp.int32),
            pltpu.VMEM((CH, d), jnp.float32),
            pltpu.VMEM((CH, d), jnp.float32),
            pltpu.SemaphoreType.DMA,
            pltpu.SemaphoreType.DMA,
            pltpu.VMEM_SHARED((n, d), jnp.float32),
        ],
    )
    def k(m_hbm, dst_hbm, out_hbm, ib0, ib1, buf0, buf1, sem0, sem1, acc):
        cid = lax.axis_index("c")
        sid = lax.axis_index("s")
        base = sid * (nch * CH)
        ibs = (ib0, ib1)
        bufs = (buf0, buf1)
        sems = (sem0, sem1)

        def fill(ref, val):
            def frow(r, carry):
                for j in range(d // L):
                    ref[r, pl.ds(j * L, L)] = jnp.full((L,), val, jnp.float32)
                return carry

            lax.fori_loop(0, CH, frow, 0)

        fill(buf0, 0.0)
        for kk in range(kmax):
            c = sid + kk * NS

            @pl.when(c < nzc)
            def _():
                pltpu.sync_copy(buf0, acc.at[pl.ds(c * zc, zc)])

        plsc.subcore_barrier()

        def fire_idx(b, c):
            pltpu.async_copy(dst_hbm.at[sid, c], ibs[b], sems[b])

        def drain_idx(b):
            pltpu.make_async_copy(dst_hbm.at[sid, 0], ibs[b], sems[b]).wait()

        @pl.when(cid == 0)
        def _():
            for b in range(2):
                fire_idx(b, b)
                pltpu.async_copy(m_hbm.at[pl.ds(base + b * CH, CH)], bufs[b],
                                 sems[b])

            def pair(kk, carry):
                for b in range(2):
                    c = 2 * kk + b
                    drain_idx(b)
                    pltpu.make_async_copy(
                        m_hbm.at[pl.ds(base, CH)], bufs[b], sems[b]).wait()
                    pltpu.sync_copy(bufs[b], acc.at[ibs[b]], add=True)

                    @pl.when(c + 2 < nch)
                    def _():
                        fire_idx(b, c + 2)
                        pltpu.async_copy(
                            m_hbm.at[pl.ds(base + (c + 2) * CH, CH)],
                            bufs[b], sems[b])
                return carry

            lax.fori_loop(0, nch // 2, pair, 0)

        @pl.when(cid == 1)
        def _():
            fill(buf1, 1.0)
            for b in range(2):
                fire_idx(b, b)

            def pair(kk, carry):
                for b in range(2):
                    c = 2 * kk + b
                    drain_idx(b)
                    pltpu.sync_copy(buf1, acc.at[ibs[b]], add=True)

                    @pl.when(c + 2 < nch)
                    def _():
                        fire_idx(b, c + 2)
                return carry

            lax.fori_loop(0, nch // 2, pair, 0)

        plsc.subcore_barrier()
        for kk in range(kmax):
            c = sid + kk * NS

            @pl.when(c < nzc)
            def _():
                pltpu.sync_copy(acc.at[pl.ds(c * zc, zc)],
                                out_hbm.at[cid, pl.ds(c * zc, zc)])

    return k(m, dst3)


def kernel(x, edge_index, edge_attr, params):
    n, d = x.shape
    E = edge_index.shape[1]
    assert d == 128 and E % (NW * CH) == 0 and n % (NS * 5) == 0

    src = edge_index[0].astype(jnp.int32)
    dst = edge_index[1].astype(jnp.int32)

    pm, pn = params["msg"], params["node"]
    W0, W1, W2 = pm["Ws"]
    b0, b1, b2 = [b.reshape(1, d) for b in pm["bs"]]
    g, bb = pm["g"].reshape(1, d), pm["b"].reshape(1, d)
    A, B, C = W0[:d], W0[d:2 * d], W0[2 * d:]

    P, Q = _tc_pre(x, A, B)

    nsl = 2
    eh = E // nsl
    parts = []
    for k in range(nsl):
        sl = slice(k * eh, (k + 1) * eh)
        src2 = src[sl].reshape(NW, eh // NW)
        dst2 = dst[sl].reshape(NW, eh // NW)
        dst3s = dst[sl].reshape(NS, (eh // NS) // CH, CH)
        S = _sc_gather_sum(P, Q, src2, dst2)
        m = _tc_edge_mlp(S, edge_attr[sl], C, b0, W1, b1, W2, b2, g, bb)
        parts.append(_sc_scatter_add(m, dst3s, n))

    U0, V1, V2 = pn["Ws"]
    c0, c1, c2 = [b.reshape(1, d) for b in pn["bs"]]
    gn, bn = pn["g"].reshape(1, d), pn["b"].reshape(1, d)
    Ua, Ub = U0[:d], U0[d:]
    x_out = _tc_node(x, parts, Ua, Ub, c0, V1, c1, V2, c2, gn, bn)
    return (x_out, edge_attr)
